# baseline (speedup 1.0000x reference)
import functools

import jax
import jax.numpy as jnp
from jax import lax
from jax.experimental import pallas as pl
from jax.experimental.pallas import tpu as pltpu
from jax.experimental.pallas import tpu_sc as plsc

_C = 5
_T = 512
_ROWS = 10240
_SC_ROWS = 6144
_NC = 2
_NS = 16
_NW = _NC * _NS
_L = 16
_RPW = _SC_ROWS // _NW
_GPW = _RPW // _L
_UNROLL = 8
_TC_BLOCK = 512


def _sc_call(vflat, tgt):
    mesh = plsc.VectorSubcoreMesh(core_axis_name="c", subcore_axis_name="s")

    @functools.partial(
        pl.kernel, mesh=mesh,
        compiler_params=pltpu.CompilerParams(needs_layout_passes=False),
        out_type=[
            jax.ShapeDtypeStruct((_SC_ROWS,), jnp.float32),
            jax.ShapeDtypeStruct((_NW * _L,), jnp.float32),
        ],
        scratch_types=[
            pltpu.VMEM((_L * _T,), jnp.float32),
            pltpu.VMEM((_L * _T,), jnp.float32),
            pltpu.VMEM((_RPW,), jnp.float32),
            pltpu.VMEM((_RPW,), jnp.float32),
            pltpu.VMEM((_L,), jnp.float32),
            pltpu.SemaphoreType.DMA,
            pltpu.SemaphoreType.DMA,
        ],
    )
    def _stca_sc(v_hbm, tgt_hbm, spike_hbm, lpart_hbm,
                 vbuf_a, vbuf_b, tgt_buf, spike_buf, loss_buf, sem_a, sem_b):
        wid = lax.axis_index("s") * _NC + lax.axis_index("c")
        base_row = wid * _RPW
        pltpu.sync_copy(tgt_hbm.at[pl.ds(base_row, _RPW)], tgt_buf)

        bufs = (vbuf_a, vbuf_b)
        sems = (sem_a, sem_b)

        def fetch(g):
            return pltpu.async_copy(
                v_hbm.at[pl.ds((base_row + g * _L) * _T, _L * _T)],
                bufs[g % 2], sems[g % 2])

        lanes = lax.iota(jnp.int32, _L)
        zero = jnp.zeros((_L,), jnp.float32)
        one = jnp.full((_L,), 1.0, jnp.float32)
        five = jnp.full((_L,), float(_C), jnp.float32)
        big = jnp.full((_L,), 1e30, jnp.float32)
        half = jnp.full((_L,), 0.5, jnp.float32)
        neg = jnp.full((_L,), -1e30, jnp.float32)
        base_idx = lanes * _T
        loss_acc = zero

        def one_step(vbuf, s):
            (idx, since, cnt, psum, pn, bc, bps, bpn, ncl, vmax) = s
            v = plsc.load_gather(vbuf, [idx])
            pos = v >= zero
            poss = v > zero
            st = pos & (since > five)
            close = st & (cnt < bc)
            bc = jnp.where(close, cnt, bc)
            bps = jnp.where(close, psum, bps)
            bpn = jnp.where(close, pn, bpn)
            inc_c = jnp.where(pos, one, zero)
            sv = jnp.where(poss, v, zero)
            inc_s = jnp.where(poss, one, zero)
            cnt = jnp.where(st, one, cnt + inc_c)
            psum = jnp.where(st, sv, psum + sv)
            pn = jnp.where(st, inc_s, pn + inc_s)
            ncl = ncl + jnp.where(st, one, zero)
            vmax = jnp.maximum(vmax, v)
            since = jnp.where(pos, one, since + one)
            return (idx + 1, since, cnt, psum, pn, bc, bps, bpn, ncl, vmax)

        def finish(s, goff):
            (_, _, cnt, psum, pn, bc, bps, bpn, ncl, vmax) = s
            close = cnt < bc
            bps = jnp.where(close, psum, bps)
            bpn = jnp.where(close, pn, bpn)
            tgtv = plsc.load_gather(tgt_buf, [goff])
            is_tgt = tgtv > half
            spiked = ncl > half
            contrib = jnp.where(bpn > zero, bps / jnp.maximum(bpn, one), zero)
            rowloss = jnp.where(is_tgt & ~spiked, -vmax,
                                jnp.where((~is_tgt) & spiked, contrib, zero))
            plsc.store_scatter(spike_buf, [goff], ncl)
            return rowloss

        pending = fetch(0)
        for g in range(_GPW):
            pending.wait()
            if g + 1 < _GPW:
                pending = fetch(g + 1)
            vbuf = bufs[g % 2]

            def step(_, s, vbuf=vbuf):
                for _u in range(_UNROLL):
                    s = one_step(vbuf, s)
                return s

            init = (base_idx, big, big, zero, zero, big, zero, zero, zero, neg)
            s_out = lax.fori_loop(0, _T // _UNROLL, step, init)
            loss_acc = loss_acc + finish(s_out, lanes + g * _L)

        loss_buf[...] = loss_acc
        pltpu.sync_copy(spike_buf, spike_hbm.at[pl.ds(base_row, _RPW)])
        pltpu.sync_copy(loss_buf, lpart_hbm.at[pl.ds(wid * _L, _L)])

    return _stca_sc(vflat, tgt)


def _cummax(x, fill):
    n = x.shape[-1]
    s = 1
    while s < n:
        pad = jnp.full(x.shape[:-1] + (s,), fill, x.dtype)
        shifted = jnp.concatenate([pad, x[..., :-s]], axis=-1)
        x = jnp.maximum(x, shifted)
        s *= 2
    return x


def _stca_tc_block(v_ref, tgt_ref, spike_ref, loss_ref, m_ref):
    T = _T

    @pl.when(pl.program_id(0) == 0)
    def _init_mats():
        a = jax.lax.broadcasted_iota(jnp.int32, (T, T), 0)
        b = jax.lax.broadcasted_iota(jnp.int32, (T, T), 1)
        m_ref[:, :T] = (a <= b).astype(jnp.bfloat16)
        m_ref[:, T:2 * T] = ((a >= b - _C) & (a <= b - 1)).astype(jnp.bfloat16)
        m_ref[:, 2 * T:] = ((a >= b + 1) & (a <= b + _C)).astype(jnp.bfloat16)
        loss_ref[...] = jnp.zeros((1, 1), jnp.float32)

    v = v_ref[...]
    R = v.shape[0]
    t_idx = jax.lax.broadcasted_iota(jnp.int32, (R, T), 1)

    is_pos = v >= 0.0
    sums = jax.lax.dot_general(
        is_pos.astype(jnp.bfloat16), m_ref[...],
        (((1,), (0,)), ((), ())), preferred_element_type=jnp.float32)
    P = sums[:, :T]
    prev_cnt = sums[:, T:2 * T]
    next_cnt = sums[:, 2 * T:]

    start = is_pos & (prev_cnt == 0.0)
    end = is_pos & (next_cnt == 0.0)

    s_ff = _cummax(jnp.where(start, P - 1.0, -1.0), jnp.float32(-1.0))
    cnt_at_end = P - s_ff

    big = jnp.float32(2 ** 30)
    t_f = t_idx.astype(jnp.float32)
    score = jnp.where(end, cnt_at_end * T + t_f, big)
    m = jnp.min(score, axis=-1)
    t_star = jnp.bitwise_and(m.astype(jnp.int32), T - 1)

    first = jnp.max(jnp.where(start & (t_idx <= t_star[:, None]), t_idx, -1),
                    axis=-1)
    span_pos = (t_idx >= first[:, None]) & (t_idx <= t_star[:, None]) & (v > 0.0)
    psum = jnp.sum(jnp.where(span_pos, v, 0.0), axis=-1)
    pcnt = jnp.sum(span_pos.astype(jnp.float32), axis=-1)
    contrib = jnp.where(pcnt > 0.0, psum / jnp.maximum(pcnt, 1.0), 0.0)

    vmax = jnp.max(v, axis=-1)
    ncl = jnp.sum(start.astype(jnp.float32), axis=-1)
    spiked = ncl > 0.0
    tgt = tgt_ref[0, 0, :] != 0
    rowloss = jnp.where(tgt & ~spiked, -vmax,
                        jnp.where((~tgt) & spiked, contrib, 0.0))

    spike_ref[0, 0, :] = ncl
    loss_ref[...] += jnp.sum(rowloss).reshape(1, 1)


def _tc_call(v2, tgt):
    rows = v2.shape[0]
    nblk = rows // _TC_BLOCK
    tgt3 = tgt.reshape(nblk, 1, _TC_BLOCK).astype(jnp.int32)
    spike, loss = pl.pallas_call(
        _stca_tc_block,
        grid=(nblk,),
        in_specs=[
            pl.BlockSpec((_TC_BLOCK, _T), lambda i: (i, 0)),
            pl.BlockSpec((1, 1, _TC_BLOCK), lambda i: (i, 0, 0)),
        ],
        out_specs=[
            pl.BlockSpec((1, 1, _TC_BLOCK), lambda i: (i, 0, 0)),
            pl.BlockSpec((1, 1), lambda i: (0, 0)),
        ],
        out_shape=[
            jax.ShapeDtypeStruct((nblk, 1, _TC_BLOCK), jnp.float32),
            jax.ShapeDtypeStruct((1, 1), jnp.float32),
        ],
        scratch_shapes=[pltpu.VMEM((_T, 3 * _T), jnp.bfloat16)],
    )(v2, tgt3)
    return spike.reshape(rows), loss[0, 0]


@jax.jit
def _run(vmem, labels):
    B, N, T = vmem.shape
    tgt = (labels[:, None] == jnp.arange(N, dtype=labels.dtype)[None, :])
    tgtf = tgt.reshape(-1).astype(jnp.float32)
    v2 = vmem.reshape(B * N, T)

    spike_sc, lpart = _sc_call(v2[:_SC_ROWS].reshape(-1), tgtf[:_SC_ROWS])
    spike_tc, loss_tc = _tc_call(v2[_SC_ROWS:], tgtf[_SC_ROWS:])

    loss = jnp.sum(lpart) + loss_tc
    spike = jnp.concatenate([spike_sc, spike_tc]).reshape(B, N)
    return loss, spike


def kernel(vmem, vlastmem, labels):
    del vlastmem
    return _run(vmem, labels)


# speedup vs baseline: 1.0906x; 1.0906x over previous
import functools

import jax
import jax.numpy as jnp
from jax import lax
from jax.experimental import pallas as pl
from jax.experimental.pallas import tpu as pltpu
from jax.experimental.pallas import tpu_sc as plsc

_C = 5
_T = 512
_ROWS = 10240
_SC_ROWS = 4608
_NC = 2
_NS = 16
_NW = _NC * _NS
_L = 16
_RPW = _SC_ROWS // _NW
_GPW = _RPW // _L
_UNROLL = 8
_TC_BLOCK = 512


def _sc_call(vflat, tgt):
    mesh = plsc.VectorSubcoreMesh(core_axis_name="c", subcore_axis_name="s")

    @functools.partial(
        pl.kernel, mesh=mesh,
        compiler_params=pltpu.CompilerParams(needs_layout_passes=False),
        out_type=[
            jax.ShapeDtypeStruct((_SC_ROWS,), jnp.float32),
            jax.ShapeDtypeStruct((_NW * _L,), jnp.float32),
        ],
        scratch_types=[
            pltpu.VMEM((_L * _T,), jnp.float32),
            pltpu.VMEM((_L * _T,), jnp.float32),
            pltpu.VMEM((_RPW,), jnp.float32),
            pltpu.VMEM((_RPW,), jnp.float32),
            pltpu.VMEM((_L,), jnp.float32),
            pltpu.SemaphoreType.DMA,
            pltpu.SemaphoreType.DMA,
        ],
    )
    def _stca_sc(v_hbm, tgt_hbm, spike_hbm, lpart_hbm,
                 vbuf_a, vbuf_b, tgt_buf, spike_buf, loss_buf, sem_a, sem_b):
        wid = lax.axis_index("s") * _NC + lax.axis_index("c")
        base_row = wid * _RPW
        pltpu.sync_copy(tgt_hbm.at[pl.ds(base_row, _RPW)], tgt_buf)

        bufs = (vbuf_a, vbuf_b)
        sems = (sem_a, sem_b)

        def fetch(g):
            return pltpu.async_copy(
                v_hbm.at[pl.ds((base_row + g * _L) * _T, _L * _T)],
                bufs[g % 2], sems[g % 2])

        lanes = lax.iota(jnp.int32, _L)
        zero = jnp.zeros((_L,), jnp.float32)
        one = jnp.full((_L,), 1.0, jnp.float32)
        five = jnp.full((_L,), float(_C), jnp.float32)
        big = jnp.full((_L,), 1e30, jnp.float32)
        half = jnp.full((_L,), 0.5, jnp.float32)
        neg = jnp.full((_L,), -1e30, jnp.float32)
        base_idx = lanes * _T
        loss_acc = zero

        def one_step(vbuf, s):
            (idx, since, cnt, psum, pn, bc, bps, bpn, ncl, vmax) = s
            v = plsc.load_gather(vbuf, [idx])
            pos = v >= zero
            poss = v > zero
            st = pos & (since > five)
            close = st & (cnt < bc)
            bc = jnp.where(close, cnt, bc)
            bps = jnp.where(close, psum, bps)
            bpn = jnp.where(close, pn, bpn)
            inc_c = jnp.where(pos, one, zero)
            sv = jnp.where(poss, v, zero)
            inc_s = jnp.where(poss, one, zero)
            cnt = jnp.where(st, one, cnt + inc_c)
            psum = jnp.where(st, sv, psum + sv)
            pn = jnp.where(st, inc_s, pn + inc_s)
            ncl = ncl + jnp.where(st, one, zero)
            vmax = jnp.maximum(vmax, v)
            since = jnp.where(pos, one, since + one)
            return (idx + 1, since, cnt, psum, pn, bc, bps, bpn, ncl, vmax)

        def finish(s, goff):
            (_, _, cnt, psum, pn, bc, bps, bpn, ncl, vmax) = s
            close = cnt < bc
            bps = jnp.where(close, psum, bps)
            bpn = jnp.where(close, pn, bpn)
            tgtv = plsc.load_gather(tgt_buf, [goff])
            is_tgt = tgtv > half
            spiked = ncl > half
            contrib = jnp.where(bpn > zero, bps / jnp.maximum(bpn, one), zero)
            rowloss = jnp.where(is_tgt & ~spiked, -vmax,
                                jnp.where((~is_tgt) & spiked, contrib, zero))
            plsc.store_scatter(spike_buf, [goff], ncl)
            return rowloss

        pending = fetch(0)
        for g in range(_GPW):
            pending.wait()
            if g + 1 < _GPW:
                pending = fetch(g + 1)
            vbuf = bufs[g % 2]

            def step(_, s, vbuf=vbuf):
                for _u in range(_UNROLL):
                    s = one_step(vbuf, s)
                return s

            init = (base_idx, big, big, zero, zero, big, zero, zero, zero, neg)
            s_out = lax.fori_loop(0, _T // _UNROLL, step, init)
            loss_acc = loss_acc + finish(s_out, lanes + g * _L)

        loss_buf[...] = loss_acc
        pltpu.sync_copy(spike_buf, spike_hbm.at[pl.ds(base_row, _RPW)])
        pltpu.sync_copy(loss_buf, lpart_hbm.at[pl.ds(wid * _L, _L)])

    return _stca_sc(vflat, tgt)


def _cummax(x, fill):
    n = x.shape[-1]
    s = 1
    while s < n:
        pad = jnp.full(x.shape[:-1] + (s,), fill, x.dtype)
        shifted = jnp.concatenate([pad, x[..., :-s]], axis=-1)
        x = jnp.maximum(x, shifted)
        s *= 2
    return x


def _stca_tc_block(v_ref, tgt_ref, spike_ref, loss_ref, m_ref):
    T = _T

    @pl.when(pl.program_id(0) == 0)
    def _init_mats():
        a = jax.lax.broadcasted_iota(jnp.int32, (T, T), 0)
        b = jax.lax.broadcasted_iota(jnp.int32, (T, T), 1)
        m_ref[:, :T] = (a <= b).astype(jnp.bfloat16)
        m_ref[:, T:2 * T] = ((a >= b - _C) & (a <= b - 1)).astype(jnp.bfloat16)
        m_ref[:, 2 * T:] = ((a >= b + 1) & (a <= b + _C)).astype(jnp.bfloat16)
        loss_ref[...] = jnp.zeros((1, 1), jnp.float32)

    v = v_ref[...]
    R = v.shape[0]
    t_idx = jax.lax.broadcasted_iota(jnp.int32, (R, T), 1)

    is_pos = v >= 0.0
    sums = jax.lax.dot_general(
        is_pos.astype(jnp.bfloat16), m_ref[...],
        (((1,), (0,)), ((), ())), preferred_element_type=jnp.float32)
    P = sums[:, :T]
    prev_cnt = sums[:, T:2 * T]
    next_cnt = sums[:, 2 * T:]

    start = is_pos & (prev_cnt == 0.0)
    end = is_pos & (next_cnt == 0.0)

    s_ff = _cummax(jnp.where(start, P - 1.0, -1.0), jnp.float32(-1.0))
    cnt_at_end = P - s_ff

    big = jnp.float32(2 ** 30)
    t_f = t_idx.astype(jnp.float32)
    score = jnp.where(end, cnt_at_end * T + t_f, big)
    m = jnp.min(score, axis=-1)
    t_star = jnp.bitwise_and(m.astype(jnp.int32), T - 1)

    first = jnp.max(jnp.where(start & (t_idx <= t_star[:, None]), t_idx, -1),
                    axis=-1)
    span_pos = (t_idx >= first[:, None]) & (t_idx <= t_star[:, None]) & (v > 0.0)
    psum = jnp.sum(jnp.where(span_pos, v, 0.0), axis=-1)
    pcnt = jnp.sum(span_pos.astype(jnp.float32), axis=-1)
    contrib = jnp.where(pcnt > 0.0, psum / jnp.maximum(pcnt, 1.0), 0.0)

    vmax = jnp.max(v, axis=-1)
    ncl = jnp.sum(start.astype(jnp.float32), axis=-1)
    spiked = ncl > 0.0
    tgt = tgt_ref[0, 0, :] != 0
    rowloss = jnp.where(tgt & ~spiked, -vmax,
                        jnp.where((~tgt) & spiked, contrib, 0.0))

    spike_ref[0, 0, :] = ncl
    loss_ref[...] += jnp.sum(rowloss).reshape(1, 1)


def _tc_call(v2, tgt):
    rows = v2.shape[0]
    nblk = rows // _TC_BLOCK
    tgt3 = tgt.reshape(nblk, 1, _TC_BLOCK).astype(jnp.int32)
    spike, loss = pl.pallas_call(
        _stca_tc_block,
        grid=(nblk,),
        in_specs=[
            pl.BlockSpec((_TC_BLOCK, _T), lambda i: (i, 0)),
            pl.BlockSpec((1, 1, _TC_BLOCK), lambda i: (i, 0, 0)),
        ],
        out_specs=[
            pl.BlockSpec((1, 1, _TC_BLOCK), lambda i: (i, 0, 0)),
            pl.BlockSpec((1, 1), lambda i: (0, 0)),
        ],
        out_shape=[
            jax.ShapeDtypeStruct((nblk, 1, _TC_BLOCK), jnp.float32),
            jax.ShapeDtypeStruct((1, 1), jnp.float32),
        ],
        scratch_shapes=[pltpu.VMEM((_T, 3 * _T), jnp.bfloat16)],
    )(v2, tgt3)
    return spike.reshape(rows), loss[0, 0]


@jax.jit
def _run(vmem, labels):
    B, N, T = vmem.shape
    tgt = (labels[:, None] == jnp.arange(N, dtype=labels.dtype)[None, :])
    tgtf = tgt.reshape(-1).astype(jnp.float32)
    v2 = vmem.reshape(B * N, T)

    spike_sc, lpart = _sc_call(v2[:_SC_ROWS].reshape(-1), tgtf[:_SC_ROWS])
    spike_tc, loss_tc = _tc_call(v2[_SC_ROWS:], tgtf[_SC_ROWS:])

    loss = jnp.sum(lpart) + loss_tc
    spike = jnp.concatenate([spike_sc, spike_tc]).reshape(B, N)
    return loss, spike


def kernel(vmem, vlastmem, labels):
    del vlastmem
    return _run(vmem, labels)


# speedup vs baseline: 1.0959x; 1.0048x over previous
import functools

import jax
import jax.numpy as jnp
from jax import lax
from jax.experimental import pallas as pl
from jax.experimental.pallas import tpu as pltpu
from jax.experimental.pallas import tpu_sc as plsc

_C = 5
_T = 512
_ROWS = 10240
_SC_ROWS = 5120
_NC = 2
_NS = 16
_NW = _NC * _NS
_L = 16
_RPW = _SC_ROWS // _NW
_GPW = _RPW // _L
_UNROLL = 8
_TC_BLOCK = 512


def _sc_call(vflat, tgt):
    mesh = plsc.VectorSubcoreMesh(core_axis_name="c", subcore_axis_name="s")

    @functools.partial(
        pl.kernel, mesh=mesh,
        compiler_params=pltpu.CompilerParams(needs_layout_passes=False),
        out_type=[
            jax.ShapeDtypeStruct((_SC_ROWS,), jnp.float32),
            jax.ShapeDtypeStruct((_NW * _L,), jnp.float32),
        ],
        scratch_types=[
            pltpu.VMEM((_L * _T,), jnp.float32),
            pltpu.VMEM((_L * _T,), jnp.float32),
            pltpu.VMEM((_RPW,), jnp.float32),
            pltpu.VMEM((_RPW,), jnp.float32),
            pltpu.VMEM((_L,), jnp.float32),
            pltpu.SemaphoreType.DMA,
            pltpu.SemaphoreType.DMA,
        ],
    )
    def _stca_sc(v_hbm, tgt_hbm, spike_hbm, lpart_hbm,
                 vbuf_a, vbuf_b, tgt_buf, spike_buf, loss_buf, sem_a, sem_b):
        wid = lax.axis_index("s") * _NC + lax.axis_index("c")
        base_row = wid * _RPW
        pltpu.sync_copy(tgt_hbm.at[pl.ds(base_row, _RPW)], tgt_buf)

        bufs = (vbuf_a, vbuf_b)
        sems = (sem_a, sem_b)

        def fetch(g):
            return pltpu.async_copy(
                v_hbm.at[pl.ds((base_row + g * _L) * _T, _L * _T)],
                bufs[g % 2], sems[g % 2])

        lanes = lax.iota(jnp.int32, _L)
        zero = jnp.zeros((_L,), jnp.float32)
        one = jnp.full((_L,), 1.0, jnp.float32)
        five = jnp.full((_L,), float(_C), jnp.float32)
        big = jnp.full((_L,), 1e30, jnp.float32)
        half = jnp.full((_L,), 0.5, jnp.float32)
        neg = jnp.full((_L,), -1e30, jnp.float32)
        base_idx = lanes * _T
        loss_acc = zero

        def one_step(vbuf, s):
            (idx, since, cnt, psum, pn, bc, bps, bpn, ncl, vmax) = s
            v = plsc.load_gather(vbuf, [idx])
            pos = v >= zero
            poss = v > zero
            st = pos & (since > five)
            close = st & (cnt < bc)
            bc = jnp.where(close, cnt, bc)
            bps = jnp.where(close, psum, bps)
            bpn = jnp.where(close, pn, bpn)
            inc_c = jnp.where(pos, one, zero)
            sv = jnp.where(poss, v, zero)
            inc_s = jnp.where(poss, one, zero)
            cnt = jnp.where(st, one, cnt + inc_c)
            psum = jnp.where(st, sv, psum + sv)
            pn = jnp.where(st, inc_s, pn + inc_s)
            ncl = ncl + jnp.where(st, one, zero)
            vmax = jnp.maximum(vmax, v)
            since = jnp.where(pos, one, since + one)
            return (idx + 1, since, cnt, psum, pn, bc, bps, bpn, ncl, vmax)

        def finish(s, goff):
            (_, _, cnt, psum, pn, bc, bps, bpn, ncl, vmax) = s
            close = cnt < bc
            bps = jnp.where(close, psum, bps)
            bpn = jnp.where(close, pn, bpn)
            tgtv = plsc.load_gather(tgt_buf, [goff])
            is_tgt = tgtv > half
            spiked = ncl > half
            contrib = jnp.where(bpn > zero, bps / jnp.maximum(bpn, one), zero)
            rowloss = jnp.where(is_tgt & ~spiked, -vmax,
                                jnp.where((~is_tgt) & spiked, contrib, zero))
            plsc.store_scatter(spike_buf, [goff], ncl)
            return rowloss

        pending = fetch(0)
        for g in range(_GPW):
            pending.wait()
            if g + 1 < _GPW:
                pending = fetch(g + 1)
            vbuf = bufs[g % 2]

            def step(_, s, vbuf=vbuf):
                for _u in range(_UNROLL):
                    s = one_step(vbuf, s)
                return s

            init = (base_idx, big, big, zero, zero, big, zero, zero, zero, neg)
            s_out = lax.fori_loop(0, _T // _UNROLL, step, init)
            loss_acc = loss_acc + finish(s_out, lanes + g * _L)

        loss_buf[...] = loss_acc
        pltpu.sync_copy(spike_buf, spike_hbm.at[pl.ds(base_row, _RPW)])
        pltpu.sync_copy(loss_buf, lpart_hbm.at[pl.ds(wid * _L, _L)])

    return _stca_sc(vflat, tgt)


def _cummax(x, fill):
    n = x.shape[-1]
    s = 1
    while s < n:
        pad = jnp.full(x.shape[:-1] + (s,), fill, x.dtype)
        shifted = jnp.concatenate([pad, x[..., :-s]], axis=-1)
        x = jnp.maximum(x, shifted)
        s *= 2
    return x


def _stca_tc_block(v_ref, tgt_ref, spike_ref, loss_ref, m_ref):
    T = _T

    @pl.when(pl.program_id(0) == 0)
    def _init_mats():
        a = jax.lax.broadcasted_iota(jnp.int32, (T, T), 0)
        b = jax.lax.broadcasted_iota(jnp.int32, (T, T), 1)
        m_ref[:, :T] = (a <= b).astype(jnp.bfloat16)
        m_ref[:, T:2 * T] = ((a >= b - _C) & (a <= b - 1)).astype(jnp.bfloat16)
        m_ref[:, 2 * T:] = ((a >= b + 1) & (a <= b + _C)).astype(jnp.bfloat16)
        loss_ref[...] = jnp.zeros((1, 1), jnp.float32)

    v = v_ref[...]
    R = v.shape[0]
    t_idx = jax.lax.broadcasted_iota(jnp.int32, (R, T), 1)

    is_pos = v >= 0.0
    sums = jax.lax.dot_general(
        is_pos.astype(jnp.bfloat16), m_ref[...],
        (((1,), (0,)), ((), ())), preferred_element_type=jnp.float32)
    P = sums[:, :T]
    prev_cnt = sums[:, T:2 * T]
    next_cnt = sums[:, 2 * T:]

    start = is_pos & (prev_cnt == 0.0)
    end = is_pos & (next_cnt == 0.0)

    s_ff = _cummax(jnp.where(start, P - 1.0, -1.0), jnp.float32(-1.0))
    cnt_at_end = P - s_ff

    big = jnp.float32(2 ** 30)
    t_f = t_idx.astype(jnp.float32)
    score = jnp.where(end, cnt_at_end * T + t_f, big)
    m = jnp.min(score, axis=-1)
    t_star = jnp.bitwise_and(m.astype(jnp.int32), T - 1)

    first = jnp.max(jnp.where(start & (t_idx <= t_star[:, None]), t_idx, -1),
                    axis=-1)
    span_pos = (t_idx >= first[:, None]) & (t_idx <= t_star[:, None]) & (v > 0.0)
    psum = jnp.sum(jnp.where(span_pos, v, 0.0), axis=-1)
    pcnt = jnp.sum(span_pos.astype(jnp.float32), axis=-1)
    contrib = jnp.where(pcnt > 0.0, psum / jnp.maximum(pcnt, 1.0), 0.0)

    vmax = jnp.max(v, axis=-1)
    ncl = jnp.sum(start.astype(jnp.float32), axis=-1)
    spiked = ncl > 0.0
    tgt = tgt_ref[0, 0, :] != 0
    rowloss = jnp.where(tgt & ~spiked, -vmax,
                        jnp.where((~tgt) & spiked, contrib, 0.0))

    spike_ref[0, 0, :] = ncl
    loss_ref[...] += jnp.sum(rowloss).reshape(1, 1)


def _tc_call(v2, tgt):
    rows = v2.shape[0]
    nblk = rows // _TC_BLOCK
    tgt3 = tgt.reshape(nblk, 1, _TC_BLOCK).astype(jnp.int32)
    spike, loss = pl.pallas_call(
        _stca_tc_block,
        grid=(nblk,),
        in_specs=[
            pl.BlockSpec((_TC_BLOCK, _T), lambda i: (i, 0)),
            pl.BlockSpec((1, 1, _TC_BLOCK), lambda i: (i, 0, 0)),
        ],
        out_specs=[
            pl.BlockSpec((1, 1, _TC_BLOCK), lambda i: (i, 0, 0)),
            pl.BlockSpec((1, 1), lambda i: (0, 0)),
        ],
        out_shape=[
            jax.ShapeDtypeStruct((nblk, 1, _TC_BLOCK), jnp.float32),
            jax.ShapeDtypeStruct((1, 1), jnp.float32),
        ],
        scratch_shapes=[pltpu.VMEM((_T, 3 * _T), jnp.bfloat16)],
    )(v2, tgt3)
    return spike.reshape(rows), loss[0, 0]


@jax.jit
def _run(vmem, labels):
    B, N, T = vmem.shape
    tgt = (labels[:, None] == jnp.arange(N, dtype=labels.dtype)[None, :])
    tgtf = tgt.reshape(-1).astype(jnp.float32)
    v2 = vmem.reshape(B * N, T)

    spike_sc, lpart = _sc_call(v2[:_SC_ROWS].reshape(-1), tgtf[:_SC_ROWS])
    spike_tc, loss_tc = _tc_call(v2[_SC_ROWS:], tgtf[_SC_ROWS:])

    loss = jnp.sum(lpart) + loss_tc
    spike = jnp.concatenate([spike_sc, spike_tc]).reshape(B, N)
    return loss, spike


def kernel(vmem, vlastmem, labels):
    del vlastmem
    return _run(vmem, labels)
